# Initial kernel scaffold; baseline (speedup 1.0000x reference)
import jax
import jax.numpy as jnp
from jax.experimental import pallas as pl


def kernel(x2, edge_index, emb_weight):
    raise NotImplementedError("write your pallas kernel here")


# speedup vs baseline: 9.2093x; 9.2093x over previous
import functools

import jax
import jax.numpy as jnp
from jax import lax
from jax.experimental import pallas as pl
from jax.experimental.pallas import tpu as pltpu
from jax.experimental.pallas import tpu_sc as plsc

N = 10000
E = 320000
D = 128
LAYERS = 3

NP = 10240
NSC = 2
NTILE = 16
NW = NSC * NTILE
EPW = E // NW
CH = 80
NCHUNK = EPW // CH
RPT = NP // NTILE

_mesh = plsc.VectorSubcoreMesh(core_axis_name="c", subcore_axis_name="s")


def _deg_body(col_hbm, degp_hbm, idx_v, ones_v, zrow_v, deg_sh):
    c = lax.axis_index("c")
    s = lax.axis_index("s")
    wid = c * NTILE + s

    for j in range(CH // 16):
        ones_v[pl.ds(j * 16, 16)] = jnp.ones((16,), jnp.float32)
    for j in range(RPT // 16):
        zrow_v[pl.ds(j * 16, 16)] = jnp.zeros((16,), jnp.float32)
    pltpu.sync_copy(zrow_v, deg_sh.at[pl.ds(s * RPT, RPT)])
    plsc.subcore_barrier()

    base = wid * EPW

    def body(i, carry):
        pltpu.sync_copy(col_hbm.at[pl.ds(base + i * CH, CH)], idx_v)
        pltpu.sync_copy(ones_v, deg_sh.at[idx_v], add=True)
        return carry

    lax.fori_loop(0, NCHUNK, body, 0)
    plsc.subcore_barrier()
    pltpu.sync_copy(deg_sh.at[pl.ds(s * RPT, RPT)],
                    degp_hbm.at[c, pl.ds(s * RPT, RPT)])


_deg_kernel = pl.kernel(
    _deg_body,
    out_type=jax.ShapeDtypeStruct((NSC, NP), jnp.float32),
    mesh=_mesh,
    scratch_types=[
        pltpu.VMEM((CH,), jnp.int32),
        pltpu.VMEM((CH,), jnp.float32),
        pltpu.VMEM((RPT,), jnp.float32),
        pltpu.VMEM_SHARED((NP,), jnp.float32),
    ],
)


def _sweep_body(z_hbm, row_hbm, col_hbm, part_hbm,
                idx_r, idx_c, rows_v, zbuf, acc_sh, sem):
    c = lax.axis_index("c")
    s = lax.axis_index("s")
    wid = c * NTILE + s

    ZB = 64

    def zb(i, carry):
        for j in range(D // 16):
            zbuf[i, pl.ds(j * 16, 16)] = jnp.zeros((16,), jnp.float32)
        return carry

    lax.fori_loop(0, ZB, zb, 0)

    def zc(i, carry):
        pltpu.sync_copy(zbuf, acc_sh.at[pl.ds(s * RPT + i * ZB, ZB)])
        return carry

    lax.fori_loop(0, RPT // ZB, zc, 0)
    plsc.subcore_barrier()

    base = wid * EPW

    def body(i, carry):
        off = base + i * CH
        pltpu.sync_copy(row_hbm.at[pl.ds(off, CH)], idx_r)
        pltpu.sync_copy(col_hbm.at[pl.ds(off, CH)], idx_c)
        pltpu.async_copy(z_hbm.at[idx_r], rows_v, sem).wait()
        pltpu.sync_copy(rows_v, acc_sh.at[idx_c], add=True)
        return carry

    lax.fori_loop(0, NCHUNK, body, 0)
    plsc.subcore_barrier()

    def wo(i, carry):
        r0 = s * RPT + i * ZB
        pltpu.sync_copy(acc_sh.at[pl.ds(r0, ZB)],
                        part_hbm.at[c, pl.ds(r0, ZB)])
        return carry

    lax.fori_loop(0, RPT // ZB, wo, 0)


_sweep_kernel = pl.kernel(
    _sweep_body,
    out_type=jax.ShapeDtypeStruct((NSC, NP, D), jnp.float32),
    mesh=_mesh,
    scratch_types=[
        pltpu.VMEM((CH,), jnp.int32),
        pltpu.VMEM((CH,), jnp.int32),
        pltpu.VMEM((CH, D), jnp.float32),
        pltpu.VMEM((64, D), jnp.float32),
        pltpu.VMEM_SHARED((NP, D), jnp.float32),
        pltpu.SemaphoreType.DMA,
    ],
)


_BLK = 512
_GRID = NP // _BLK


def _scale_body(degp_ref, emb_ref, dis_ref, z0_ref):
    deg = degp_ref[0] + degp_ref[1]
    dis = jnp.where(deg > 0, lax.rsqrt(jnp.maximum(deg, 1.0)), 0.0)
    dis_ref[...] = dis
    z0_ref[...] = dis * emb_ref[...]


_scale_kernel = pl.pallas_call(
    _scale_body,
    grid=(_GRID,),
    in_specs=[
        pl.BlockSpec((NSC, _BLK, 1), lambda i: (0, i, 0)),
        pl.BlockSpec((_BLK, D), lambda i: (i, 0)),
    ],
    out_specs=[
        pl.BlockSpec((_BLK, 1), lambda i: (i, 0)),
        pl.BlockSpec((_BLK, D), lambda i: (i, 0)),
    ],
    out_shape=[
        jax.ShapeDtypeStruct((NP, 1), jnp.float32),
        jax.ShapeDtypeStruct((NP, D), jnp.float32),
    ],
)


def _combine_body(part_ref, dis_ref, osum_ref, z_ref, osum_out_ref):
    acc = part_ref[0] + part_ref[1]
    dis = dis_ref[...]
    da = dis * acc
    z_ref[...] = dis * da
    osum_out_ref[...] = osum_ref[...] + da * (1.0 / LAYERS)


_combine_kernel = pl.pallas_call(
    _combine_body,
    grid=(_GRID,),
    in_specs=[
        pl.BlockSpec((NSC, _BLK, D), lambda i: (0, i, 0)),
        pl.BlockSpec((_BLK, 1), lambda i: (i, 0)),
        pl.BlockSpec((_BLK, D), lambda i: (i, 0)),
    ],
    out_specs=[
        pl.BlockSpec((_BLK, D), lambda i: (i, 0)),
        pl.BlockSpec((_BLK, D), lambda i: (i, 0)),
    ],
    out_shape=[
        jax.ShapeDtypeStruct((NP, D), jnp.float32),
        jax.ShapeDtypeStruct((NP, D), jnp.float32),
    ],
)


def kernel(x2, edge_index, emb_weight):
    del x2
    row = edge_index[0].astype(jnp.int32)
    col = edge_index[1].astype(jnp.int32)
    emb_pad = jnp.zeros((NP, D), jnp.float32).at[:N].set(emb_weight)

    degp = _deg_kernel(col)
    degp = degp.reshape(NSC, NP, 1)
    dis, z = _scale_kernel(degp, emb_pad)

    osum = jnp.zeros((NP, D), jnp.float32)
    for _ in range(LAYERS):
        part = _sweep_kernel(z, row, col)
        z, osum = _combine_kernel(part, dis, osum)

    return osum[:N]


# speedup vs baseline: 21.4517x; 2.3293x over previous
import functools

import jax
import jax.numpy as jnp
from jax import lax
from jax.experimental import pallas as pl
from jax.experimental.pallas import tpu as pltpu
from jax.experimental.pallas import tpu_sc as plsc

N = 10000
E = 320000
D = 128
LAYERS = 3

NP = 10240
NSC = 2
NTILE = 16
NW = NSC * NTILE
EPW = E // NW
CH = 80
NCHUNK = EPW // CH
DH = D // NSC
NCHT = E // CH // NTILE
RPT = NP // NTILE

_mesh = plsc.VectorSubcoreMesh(core_axis_name="c", subcore_axis_name="s")


def _deg_body(col2_hbm, degp_hbm, cidx_v, ones_v, zrow_v, deg_sh, sem):
    c = lax.axis_index("c")
    s = lax.axis_index("s")
    wid = c * NTILE + s

    for j in range(CH // 16):
        ones_v[pl.ds(j * 16, 16)] = jnp.ones((16,), jnp.float32)
    for j in range(RPT // 16):
        zrow_v[pl.ds(j * 16, 16)] = jnp.zeros((16,), jnp.float32)
    pltpu.sync_copy(zrow_v, deg_sh.at[pl.ds(s * RPT, RPT)])
    pltpu.sync_copy(col2_hbm.at[wid], cidx_v)
    plsc.subcore_barrier()

    K = 25

    def batch(b, carry):
        def fire(i, carry2):
            pltpu.async_copy(ones_v, deg_sh.at[cidx_v.at[b * K + i]], sem,
                             add=True)
            return carry2

        lax.fori_loop(0, K, fire, 0)

        def drain(i, carry2):
            pltpu.make_async_copy(
                ones_v, deg_sh.at[cidx_v.at[0]], sem).wait()
            return carry2

        lax.fori_loop(0, K, drain, 0)
        return carry

    lax.fori_loop(0, NCHUNK // K, batch, 0)
    plsc.subcore_barrier()
    pltpu.sync_copy(deg_sh.at[pl.ds(s * RPT, RPT)],
                    degp_hbm.at[c, pl.ds(s * RPT, RPT)])


_deg_kernel = pl.kernel(
    _deg_body,
    out_type=jax.ShapeDtypeStruct((NSC, NP), jnp.float32),
    mesh=_mesh,
    scratch_types=[
        pltpu.VMEM((NCHUNK, CH), jnp.int32),
        pltpu.VMEM((CH,), jnp.float32),
        pltpu.VMEM((RPT,), jnp.float32),
        pltpu.VMEM_SHARED((NP,), jnp.float32),
        pltpu.SemaphoreType.DMA,
    ],
)


NB = 5


def _sweep_body(zst_hbm, row16_hbm, col16_hbm, part_hbm,
                ridx, cidx, r0, r1, r2, r3, r4, zbuf, acc_sh,
                g0, g1, g2, g3, g4, s0, s1, s2, s3, s4):
    c = lax.axis_index("c")
    s = lax.axis_index("s")

    rows = [r0, r1, r2, r3, r4]
    gsem = [g0, g1, g2, g3, g4]
    ssem = [s0, s1, s2, s3, s4]

    zsrc = zst_hbm.at[c]

    pltpu.sync_copy(row16_hbm.at[s], ridx)
    pltpu.sync_copy(col16_hbm.at[s], cidx)

    ZB = 64

    def zb(i, carry):
        for j in range(DH // 16):
            zbuf[i, pl.ds(j * 16, 16)] = jnp.zeros((16,), jnp.float32)
        return carry

    lax.fori_loop(0, ZB, zb, 0)

    def zc(i, carry):
        pltpu.sync_copy(zbuf, acc_sh.at[pl.ds(s * RPT + i * ZB, ZB)])
        return carry

    lax.fori_loop(0, RPT // ZB, zc, 0)
    plsc.subcore_barrier()

    def gather(i, b):
        pltpu.async_copy(zsrc.at[ridx.at[i]], rows[b], gsem[b])

    def gwait(b):
        pltpu.make_async_copy(zsrc.at[ridx.at[0]], rows[b], gsem[b]).wait()

    def scat(i, b):
        pltpu.async_copy(rows[b], acc_sh.at[cidx.at[i]], ssem[b], add=True)

    def swait(b):
        pltpu.make_async_copy(rows[b], acc_sh.at[cidx.at[0]], ssem[b]).wait()

    for b in range(NB - 1):
        gather(b, b)

    def step(go, first, last):
        g = go * NB
        for b in range(NB):
            i = g + b
            gwait(b)
            scat(i, b)
            b4 = (b + NB - 1) % NB
            j = i + NB - 1
            if last and b != 0:
                continue
            if not (first and b == 0):
                swait(b4)
            gather(j, b4)

    step(0, True, False)

    def mid(go, carry):
        step(go, False, False)
        return carry

    lax.fori_loop(1, NCHT // NB - 1, mid, 0)
    step(NCHT // NB - 1, False, True)

    for b in range(NB):
        swait(b)
    plsc.subcore_barrier()

    def wo(i, carry):
        rr = s * RPT + i * ZB
        pltpu.sync_copy(acc_sh.at[pl.ds(rr, ZB)],
                        part_hbm.at[c, pl.ds(rr, ZB)])
        return carry

    lax.fori_loop(0, RPT // ZB, wo, 0)


_sweep_kernel = pl.kernel(
    _sweep_body,
    out_type=jax.ShapeDtypeStruct((NSC, NP, DH), jnp.float32),
    mesh=_mesh,
    scratch_types=(
        [pltpu.VMEM((NCHT, CH), jnp.int32)] * 2
        + [pltpu.VMEM((CH, DH), jnp.float32)] * NB
        + [pltpu.VMEM((64, DH), jnp.float32),
           pltpu.VMEM_SHARED((NP, DH), jnp.float32)]
        + [pltpu.SemaphoreType.DMA] * (2 * NB)
    ),
    compiler_params=pltpu.CompilerParams(use_tc_tiling_on_sc=False),
)


_BLK = 512
_GRID = NP // _BLK


def _scale_body(degp_ref, emb_ref, dis_ref, z0_ref):
    deg = degp_ref[0] + degp_ref[1]
    dis = jnp.where(deg > 0, lax.rsqrt(jnp.maximum(deg, 1.0)), 0.0)
    dis_ref[...] = dis
    z0_ref[0] = dis * emb_ref[:, :DH]
    z0_ref[1] = dis * emb_ref[:, DH:]


_scale_kernel = pl.pallas_call(
    _scale_body,
    grid=(_GRID,),
    in_specs=[
        pl.BlockSpec((NSC, _BLK, 1), lambda i: (0, i, 0)),
        pl.BlockSpec((_BLK, D), lambda i: (i, 0)),
    ],
    out_specs=[
        pl.BlockSpec((_BLK, 1), lambda i: (i, 0)),
        pl.BlockSpec((NSC, _BLK, DH), lambda i: (0, i, 0)),
    ],
    out_shape=[
        jax.ShapeDtypeStruct((NP, 1), jnp.float32),
        jax.ShapeDtypeStruct((NSC, NP, DH), jnp.float32),
    ],
)


def _combine_body(part_ref, dis_ref, osum_ref, z_ref, osum_out_ref):
    dis = dis_ref[...]
    da0 = dis * part_ref[0]
    da1 = dis * part_ref[1]
    z_ref[0] = dis * da0
    z_ref[1] = dis * da1
    osum_out_ref[:, :DH] = osum_ref[:, :DH] + da0 * (1.0 / LAYERS)
    osum_out_ref[:, DH:] = osum_ref[:, DH:] + da1 * (1.0 / LAYERS)


_combine_kernel = pl.pallas_call(
    _combine_body,
    grid=(_GRID,),
    in_specs=[
        pl.BlockSpec((NSC, _BLK, DH), lambda i: (0, i, 0)),
        pl.BlockSpec((_BLK, 1), lambda i: (i, 0)),
        pl.BlockSpec((_BLK, D), lambda i: (i, 0)),
    ],
    out_specs=[
        pl.BlockSpec((NSC, _BLK, DH), lambda i: (0, i, 0)),
        pl.BlockSpec((_BLK, D), lambda i: (i, 0)),
    ],
    out_shape=[
        jax.ShapeDtypeStruct((NSC, NP, DH), jnp.float32),
        jax.ShapeDtypeStruct((NP, D), jnp.float32),
    ],
)


def kernel(x2, edge_index, emb_weight):
    del x2
    col32 = edge_index[1].astype(jnp.int32).reshape(NW, NCHUNK, CH)
    row16 = edge_index[0].astype(jnp.int32).reshape(NTILE, NCHT, CH)
    col16 = edge_index[1].astype(jnp.int32).reshape(NTILE, NCHT, CH)
    emb_pad = jnp.zeros((NP, D), jnp.float32).at[:N].set(emb_weight)

    degp = _deg_kernel(col32)
    degp = degp.reshape(NSC, NP, 1)
    dis, z = _scale_kernel(degp, emb_pad)

    osum = jnp.zeros((NP, D), jnp.float32)
    for _ in range(LAYERS):
        part = _sweep_kernel(z, row16, col16)
        z, osum = _combine_kernel(part, dis, osum)

    return osum[:N]
